# baseline (speedup 1.0000x reference)
import functools

import jax
import jax.numpy as jnp
from jax import lax
from jax.experimental import pallas as pl
from jax.experimental.pallas import tpu as pltpu
from jax.experimental.pallas import tpu_sc as plsc

N = 10000
E = 320000
NW = 32
C = 128
NCHT = E // C
RPT = N // 16

_mesh = plsc.VectorSubcoreMesh(core_axis_name="c", subcore_axis_name="s")


@functools.partial(
    pl.kernel,
    out_type=jax.ShapeDtypeStruct((E, 64), jnp.float32),
    mesh=_mesh,
    scratch_types=[
        pltpu.VMEM((C,), jnp.int32),
        pltpu.VMEM((C,), jnp.int32),
        pltpu.VMEM((C, 64), jnp.float32),
        pltpu.VMEM((C, 64), jnp.float32),
        pltpu.SemaphoreType.DMA,
        pltpu.SemaphoreType.DMA,
    ],
    compiler_params=pltpu.CompilerParams(use_tc_tiling_on_sc=False),
)
def _gather_add(dst_h, src_h, pd_h, ps_h, g_h, idxd, idxs, bufa, bufb, sem0, sem1):
    c = lax.axis_index("c")
    s = lax.axis_index("s")
    w = c * 16 + s
    nch = (NCHT + NW - 1 - w) // NW

    def chunk(k, carry):
        base = (w + k * NW) * C
        pltpu.sync_copy(dst_h.at[pl.ds(base, C)], idxd)
        pltpu.sync_copy(src_h.at[pl.ds(base, C)], idxs)
        cpa = pltpu.async_copy(pd_h.at[idxd], bufa, sem0)
        cpb = pltpu.async_copy(ps_h.at[idxs], bufb, sem1)
        cpa.wait()
        cpb.wait()

        def row(i, carry2):
            for j in range(4):
                sl = pl.ds(j * 16, 16)
                bufa[i, sl] = bufa[i, sl] + bufb[i, sl]
            return carry2

        lax.fori_loop(0, C, row, 0, unroll=4)
        pltpu.sync_copy(bufa, g_h.at[pl.ds(base, C)])
        return carry

    lax.fori_loop(0, nch, chunk, 0)


@functools.partial(
    pl.kernel,
    out_type=jax.ShapeDtypeStruct((2, N, 64), jnp.float32),
    mesh=_mesh,
    scratch_types=[
        pltpu.VMEM((C,), jnp.int32),
        pltpu.VMEM((C, 64), jnp.float32),
        pltpu.VMEM_SHARED((N, 64), jnp.float32),
        pltpu.SemaphoreType.DMA,
    ],
    compiler_params=pltpu.CompilerParams(use_tc_tiling_on_sc=False),
)
def _segsum(dst_h, val_h, zeros_h, out_h, idx, buf, accum, sem):
    c = lax.axis_index("c")
    s = lax.axis_index("s")
    rsl = pl.ds(s * RPT, RPT)
    pltpu.sync_copy(zeros_h.at[rsl], accum.at[rsl])
    plsc.subcore_barrier()
    w = c * 16 + s
    nch = (NCHT + NW - 1 - w) // NW

    def chunk(k, carry):
        base = (w + k * NW) * C
        pltpu.sync_copy(dst_h.at[pl.ds(base, C)], idx)
        pltpu.sync_copy(val_h.at[pl.ds(base, C)], buf)
        pltpu.sync_copy(buf, accum.at[idx], add=True)
        return carry

    lax.fori_loop(0, nch, chunk, 0)
    plsc.subcore_barrier()
    pltpu.sync_copy(accum.at[rsl], out_h.at[c, rsl])


@functools.partial(
    pl.kernel,
    out_type=jax.ShapeDtypeStruct((2, N, 16), jnp.float32),
    mesh=_mesh,
    scratch_types=[
        pltpu.VMEM((C,), jnp.int32),
        pltpu.VMEM((C, 16), jnp.float32),
        pltpu.VMEM_SHARED((N, 16), jnp.float32),
        pltpu.SemaphoreType.DMA,
    ],
    compiler_params=pltpu.CompilerParams(use_tc_tiling_on_sc=False),
)
def _degree(dst_h, ones_h, zeros_h, out_h, idx, buf, accum, sem):
    c = lax.axis_index("c")
    s = lax.axis_index("s")
    rsl = pl.ds(s * RPT, RPT)
    pltpu.sync_copy(zeros_h.at[rsl], accum.at[rsl])
    pltpu.sync_copy(ones_h, buf)
    plsc.subcore_barrier()
    w = c * 16 + s
    nch = (NCHT + NW - 1 - w) // NW

    def chunk(k, carry):
        base = (w + k * NW) * C
        pltpu.sync_copy(dst_h.at[pl.ds(base, C)], idx)
        pltpu.sync_copy(buf, accum.at[idx], add=True)
        return carry

    lax.fori_loop(0, nch, chunk, 0)
    plsc.subcore_barrier()
    pltpu.sync_copy(accum.at[rsl], out_h.at[c, rsl])


TILE_N = 2000
TILE_E = 4000


def _tables_body(x_ref, wd_ref, ws_ref, bd_ref, pd_ref, ps_ref):
    xb = x_ref[...]
    pd_ref[...] = jnp.dot(xb, wd_ref[...], preferred_element_type=jnp.float32) + bd_ref[...]
    ps_ref[...] = jnp.dot(xb, ws_ref[...], preferred_element_type=jnp.float32)


def _mlp_body(g_ref, w1_ref, b1_ref, w2_ref, b2_ref, o_ref):
    h1 = jnp.maximum(g_ref[...], 0.0)
    h2 = jnp.maximum(
        jnp.dot(h1, w1_ref[...], preferred_element_type=jnp.float32) + b1_ref[...], 0.0)
    o_ref[...] = jnp.maximum(
        jnp.dot(h2, w2_ref[...], preferred_element_type=jnp.float32) + b2_ref[...], 0.0)


def _mid_body(sp_ref, degp_ref, w3_ref, b3_ref, wd2_ref, ws2_ref, bd2_ref,
              pd2_ref, ps2_ref):
    ssum = sp_ref[0] + sp_ref[1]
    deg = degp_ref[0, :, 0:1] + degp_ref[1, :, 0:1]
    h = jnp.maximum(
        jnp.dot(ssum, w3_ref[...], preferred_element_type=jnp.float32) + deg * b3_ref[...],
        0.0)
    pd2_ref[...] = jnp.dot(h, wd2_ref[...], preferred_element_type=jnp.float32) + bd2_ref[...]
    ps2_ref[...] = jnp.dot(h, ws2_ref[...], preferred_element_type=jnp.float32)


def _final_body(sp_ref, degp_ref, w3_ref, b3_ref, o_ref):
    ssum = sp_ref[0] + sp_ref[1]
    deg = degp_ref[0, :, 0:1] + degp_ref[1, :, 0:1]
    o_ref[...] = jnp.dot(ssum, w3_ref[...], preferred_element_type=jnp.float32) + deg * b3_ref[...]


def _tables(x, wd, ws, bd):
    din = x.shape[1]
    return pl.pallas_call(
        _tables_body,
        grid=(N // TILE_N,),
        in_specs=[
            pl.BlockSpec((TILE_N, din), lambda i: (i, 0)),
            pl.BlockSpec((din, 64), lambda i: (0, 0)),
            pl.BlockSpec((din, 64), lambda i: (0, 0)),
            pl.BlockSpec((1, 64), lambda i: (0, 0)),
        ],
        out_specs=[
            pl.BlockSpec((TILE_N, 64), lambda i: (i, 0)),
            pl.BlockSpec((TILE_N, 64), lambda i: (i, 0)),
        ],
        out_shape=[
            jax.ShapeDtypeStruct((N, 64), jnp.float32),
            jax.ShapeDtypeStruct((N, 64), jnp.float32),
        ],
    )(x, wd, ws, bd)


def _mlp(g, w1, b1, w2, b2):
    return pl.pallas_call(
        _mlp_body,
        grid=(E // TILE_E,),
        in_specs=[
            pl.BlockSpec((TILE_E, 64), lambda i: (i, 0)),
            pl.BlockSpec((64, 64), lambda i: (0, 0)),
            pl.BlockSpec((1, 64), lambda i: (0, 0)),
            pl.BlockSpec((64, 64), lambda i: (0, 0)),
            pl.BlockSpec((1, 64), lambda i: (0, 0)),
        ],
        out_specs=pl.BlockSpec((TILE_E, 64), lambda i: (i, 0)),
        out_shape=jax.ShapeDtypeStruct((E, 64), jnp.float32),
    )(g, w1, b1, w2, b2)


def _mid(sp, degp, w3, b3, wd2, ws2, bd2):
    return pl.pallas_call(
        _mid_body,
        grid=(N // TILE_N,),
        in_specs=[
            pl.BlockSpec((2, TILE_N, 64), lambda i: (0, i, 0)),
            pl.BlockSpec((2, TILE_N, 16), lambda i: (0, i, 0)),
            pl.BlockSpec((64, 64), lambda i: (0, 0)),
            pl.BlockSpec((1, 64), lambda i: (0, 0)),
            pl.BlockSpec((64, 64), lambda i: (0, 0)),
            pl.BlockSpec((64, 64), lambda i: (0, 0)),
            pl.BlockSpec((1, 64), lambda i: (0, 0)),
        ],
        out_specs=[
            pl.BlockSpec((TILE_N, 64), lambda i: (i, 0)),
            pl.BlockSpec((TILE_N, 64), lambda i: (i, 0)),
        ],
        out_shape=[
            jax.ShapeDtypeStruct((N, 64), jnp.float32),
            jax.ShapeDtypeStruct((N, 64), jnp.float32),
        ],
    )(sp, degp, w3, b3, wd2, ws2, bd2)


def _final(sp, degp, w3, b3):
    return pl.pallas_call(
        _final_body,
        grid=(N // TILE_N,),
        in_specs=[
            pl.BlockSpec((2, TILE_N, 64), lambda i: (0, i, 0)),
            pl.BlockSpec((2, TILE_N, 16), lambda i: (0, i, 0)),
            pl.BlockSpec((64, 128), lambda i: (0, 0)),
            pl.BlockSpec((1, 128), lambda i: (0, 0)),
        ],
        out_specs=pl.BlockSpec((TILE_N, 128), lambda i: (i, 0)),
        out_shape=jax.ShapeDtypeStruct((N, 128), jnp.float32),
    )(sp, degp, w3, b3)


def kernel(x, edge_index, params):
    src = edge_index[0]
    dst = edge_index[1]
    (W0, b0), (W1, b1), (W2, b2), (W3, b3) = params["conv1"]
    (V0, c0), (V1, c1), (V2, c2), (V3, c3) = params["conv2"]
    W0i, W0j = W0[:128], W0[128:]
    V0i, V0j = V0[:64], V0[64:]

    zeros64 = jnp.zeros((N, 64), jnp.float32)
    zeros16 = jnp.zeros((N, 16), jnp.float32)
    onesC = jnp.ones((C, 16), jnp.float32)

    degp = _degree(dst, onesC, zeros16)

    pd1, ps1 = _tables(x, W0i - W0j, W0j, b0.reshape(1, 64))
    g1 = _gather_add(dst, src, pd1, ps1)
    h3_1 = _mlp(g1, W1, b1.reshape(1, 64), W2, b2.reshape(1, 64))
    s1p = _segsum(dst, h3_1, zeros64)

    pd2, ps2 = _mid(s1p, degp, W3, b3.reshape(1, 64),
                    V0i - V0j, V0j, c0.reshape(1, 64))
    g2 = _gather_add(dst, src, pd2, ps2)
    h3_2 = _mlp(g2, V1, c1.reshape(1, 64), V2, c2.reshape(1, 64))
    s2p = _segsum(dst, h3_2, zeros64)

    return _final(s2p, degp, V3, c3.reshape(1, 128))


# speedup vs baseline: 1.3122x; 1.3122x over previous
import functools

import jax
import jax.numpy as jnp
from jax import lax
from jax.experimental import pallas as pl
from jax.experimental.pallas import tpu as pltpu
from jax.experimental.pallas import tpu_sc as plsc

N = 10000
E = 320000
NW = 32
C = 128
NCH = 78
NEXTRA = E // C - NCH * NW
RPT = N // 16

_mesh = plsc.VectorSubcoreMesh(core_axis_name="c", subcore_axis_name="s")


@functools.partial(
    pl.kernel,
    out_type=jax.ShapeDtypeStruct((E, 64), jnp.float32),
    mesh=_mesh,
    scratch_types=[
        pltpu.VMEM((2, C), jnp.int32),
        pltpu.VMEM((2, C), jnp.int32),
        pltpu.VMEM((2, C, 64), jnp.float32),
        pltpu.VMEM((2, C, 64), jnp.float32),
        pltpu.SemaphoreType.DMA,
        pltpu.SemaphoreType.DMA,
        pltpu.SemaphoreType.DMA,
        pltpu.SemaphoreType.DMA,
        pltpu.SemaphoreType.DMA,
        pltpu.SemaphoreType.DMA,
    ],
    compiler_params=pltpu.CompilerParams(use_tc_tiling_on_sc=False),
)
def _gather_add(dst_h, src_h, pd_h, ps_h, g_h, idxd, idxs, bufa, bufb,
                semi0, semi1, semg0, semg1, semo0, semo1):
    c = lax.axis_index("c")
    s = lax.axis_index("s")
    w = c * 16 + s
    tbase = w * (NCH * C)
    semi = [semi0, semi1]
    semg = [semg0, semg1]
    semo = [semo0, semo1]

    def issue_idx(k, b):
        base = tbase + k * C
        pltpu.async_copy(dst_h.at[pl.ds(base, C)], idxd.at[b], semi[b])
        pltpu.async_copy(src_h.at[pl.ds(base, C)], idxs.at[b], semi[b])

    def wait_idx(b):
        pltpu.make_async_copy(dst_h.at[pl.ds(0, C)], idxd.at[b], semi[b]).wait()
        pltpu.make_async_copy(src_h.at[pl.ds(0, C)], idxs.at[b], semi[b]).wait()

    def issue_gather(b):
        pltpu.async_copy(pd_h.at[idxd.at[b]], bufa.at[b], semg[b])
        pltpu.async_copy(ps_h.at[idxs.at[b]], bufb.at[b], semg[b])

    def wait_gather(b):
        pltpu.make_async_copy(pd_h.at[idxd.at[b]], bufa.at[b], semg[b]).wait()
        pltpu.make_async_copy(ps_h.at[idxs.at[b]], bufb.at[b], semg[b]).wait()

    def issue_store(k, b):
        pltpu.async_copy(bufa.at[b], g_h.at[pl.ds(tbase + k * C, C)], semo[b])

    def wait_store(b):
        pltpu.make_async_copy(bufa.at[b], g_h.at[pl.ds(tbase, C)], semo[b]).wait()

    def add_rows(b, nrows):
        def row(i, carry):
            for j in range(4):
                sl = pl.ds(j * 16, 16)
                bufa[b, i, sl] = bufa[b, i, sl] + bufb[b, i, sl]
            return carry
        lax.fori_loop(0, nrows, row, 0, unroll=4)

    issue_idx(0, 0)
    issue_idx(1, 1)
    wait_idx(0)
    issue_gather(0)
    wait_gather(0)
    issue_idx(2, 0)
    wait_idx(1)
    issue_gather(1)
    add_rows(0, C)
    issue_store(0, 0)
    wait_gather(1)
    issue_idx(3, 1)
    wait_idx(0)
    wait_store(0)
    issue_gather(0)
    add_rows(1, C)
    issue_store(1, 1)

    def pair(kk, carry):
        for b in (0, 1):
            k = 2 + 2 * kk + b
            o = 1 - b
            wait_gather(b)
            issue_idx(k + 2, b)
            wait_idx(o)
            wait_store(o)
            issue_gather(o)
            add_rows(b, C)
            issue_store(k, b)
        return carry

    lax.fori_loop(0, (NCH - 4) // 2, pair, 0)
    wait_gather(0)
    wait_idx(1)
    wait_store(1)
    issue_gather(1)
    add_rows(0, C)
    issue_store(NCH - 2, 0)
    wait_gather(1)
    add_rows(1, C)
    issue_store(NCH - 1, 1)
    wait_store(0)
    wait_store(1)

    @pl.when(w < NEXTRA)
    def _():
        base = (NCH * NW + w) * C
        pltpu.sync_copy(dst_h.at[pl.ds(base, C)], idxd.at[0])
        pltpu.sync_copy(src_h.at[pl.ds(base, C)], idxs.at[0])
        cpa = pltpu.async_copy(pd_h.at[idxd.at[0]], bufa.at[0], semg[0])
        cpb = pltpu.async_copy(ps_h.at[idxs.at[0]], bufb.at[0], semg[0])
        cpa.wait()
        cpb.wait()
        add_rows(0, C)
        pltpu.sync_copy(bufa.at[0], g_h.at[pl.ds(base, C)])


@functools.partial(
    pl.kernel,
    out_type=jax.ShapeDtypeStruct((2, N, 64), jnp.float32),
    mesh=_mesh,
    scratch_types=[
        pltpu.VMEM((2, C), jnp.int32),
        pltpu.VMEM((2, C, 64), jnp.float32),
        pltpu.VMEM_SHARED((N, 64), jnp.float32),
        pltpu.SemaphoreType.DMA,
        pltpu.SemaphoreType.DMA,
        pltpu.SemaphoreType.DMA,
        pltpu.SemaphoreType.DMA,
    ],
    compiler_params=pltpu.CompilerParams(use_tc_tiling_on_sc=False),
)
def _segsum(dst_h, val_h, zeros_h, out_h, idx, buf, accum,
            seml0, seml1, semc0, semc1):
    c = lax.axis_index("c")
    s = lax.axis_index("s")
    rsl = pl.ds(s * RPT, RPT)
    pltpu.sync_copy(zeros_h.at[rsl], accum.at[rsl])
    plsc.subcore_barrier()
    w = c * 16 + s
    tbase = w * (NCH * C)
    seml = [seml0, seml1]
    semc = [semc0, semc1]

    def issue_load(k, b):
        base = tbase + k * C
        pltpu.async_copy(dst_h.at[pl.ds(base, C)], idx.at[b], seml[b])
        pltpu.async_copy(val_h.at[pl.ds(base, C)], buf.at[b], seml[b])

    def wait_load(b):
        pltpu.make_async_copy(dst_h.at[pl.ds(0, C)], idx.at[b], seml[b]).wait()
        pltpu.make_async_copy(val_h.at[pl.ds(0, C)], buf.at[b], seml[b]).wait()

    def issue_scat(b):
        pltpu.async_copy(buf.at[b], accum.at[idx.at[b]], semc[b], add=True)

    def wait_scat(b):
        pltpu.make_async_copy(buf.at[b], accum.at[idx.at[b]], semc[b]).wait()

    issue_load(0, 0)
    wait_load(0)
    issue_scat(0)
    issue_load(1, 1)

    def pair(kk, carry):
        for b in (1, 0):
            k = 1 + 2 * kk + (1 - b)
            o = 1 - b
            wait_load(b)
            issue_scat(b)
            wait_scat(o)
            issue_load(k + 1, o)
        return carry

    lax.fori_loop(0, (NCH - 2) // 2, pair, 0)
    wait_load(1)
    issue_scat(1)
    wait_scat(0)
    wait_scat(1)

    @pl.when(w < NEXTRA)
    def _():
        base = (NCH * NW + w) * C
        pltpu.sync_copy(dst_h.at[pl.ds(base, C)], idx.at[0])
        pltpu.sync_copy(val_h.at[pl.ds(base, C)], buf.at[0])
        pltpu.sync_copy(buf.at[0], accum.at[idx.at[0]], add=True)

    plsc.subcore_barrier()
    pltpu.sync_copy(accum.at[rsl], out_h.at[c, rsl])


@functools.partial(
    pl.kernel,
    out_type=jax.ShapeDtypeStruct((2, N, 16), jnp.float32),
    mesh=_mesh,
    scratch_types=[
        pltpu.VMEM((2, C), jnp.int32),
        pltpu.VMEM((C, 16), jnp.float32),
        pltpu.VMEM_SHARED((N, 16), jnp.float32),
        pltpu.SemaphoreType.DMA,
        pltpu.SemaphoreType.DMA,
        pltpu.SemaphoreType.DMA,
        pltpu.SemaphoreType.DMA,
    ],
    compiler_params=pltpu.CompilerParams(use_tc_tiling_on_sc=False),
)
def _degree(dst_h, ones_h, zeros_h, out_h, idx, buf, accum,
            seml0, seml1, semc0, semc1):
    c = lax.axis_index("c")
    s = lax.axis_index("s")
    rsl = pl.ds(s * RPT, RPT)
    pltpu.sync_copy(zeros_h.at[rsl], accum.at[rsl])
    pltpu.sync_copy(ones_h, buf)
    plsc.subcore_barrier()
    w = c * 16 + s
    tbase = w * (NCH * C)
    seml = [seml0, seml1]
    semc = [semc0, semc1]

    def issue_load(k, b):
        pltpu.async_copy(dst_h.at[pl.ds(tbase + k * C, C)], idx.at[b], seml[b])

    def wait_load(b):
        pltpu.make_async_copy(dst_h.at[pl.ds(0, C)], idx.at[b], seml[b]).wait()

    def issue_scat(b):
        pltpu.async_copy(buf, accum.at[idx.at[b]], semc[b], add=True)

    def wait_scat(b):
        pltpu.make_async_copy(buf, accum.at[idx.at[b]], semc[b]).wait()

    issue_load(0, 0)
    wait_load(0)
    issue_scat(0)
    issue_load(1, 1)

    def pair(kk, carry):
        for b in (1, 0):
            k = 1 + 2 * kk + (1 - b)
            o = 1 - b
            wait_load(b)
            issue_scat(b)
            wait_scat(o)
            issue_load(k + 1, o)
        return carry

    lax.fori_loop(0, (NCH - 2) // 2, pair, 0)
    wait_load(1)
    issue_scat(1)
    wait_scat(0)
    wait_scat(1)

    @pl.when(w < NEXTRA)
    def _():
        base = (NCH * NW + w) * C
        pltpu.sync_copy(dst_h.at[pl.ds(base, C)], idx.at[0])
        pltpu.sync_copy(buf, accum.at[idx.at[0]], add=True)

    plsc.subcore_barrier()
    pltpu.sync_copy(accum.at[rsl], out_h.at[c, rsl])


TILE_N = 2000
TILE_E = 4000


def _tables_body(x_ref, wd_ref, ws_ref, bd_ref, pd_ref, ps_ref):
    xb = x_ref[...]
    pd_ref[...] = jnp.dot(xb, wd_ref[...], preferred_element_type=jnp.float32) + bd_ref[...]
    ps_ref[...] = jnp.dot(xb, ws_ref[...], preferred_element_type=jnp.float32)


def _mlp_body(g_ref, w1_ref, b1_ref, w2_ref, b2_ref, o_ref):
    h1 = jnp.maximum(g_ref[...], 0.0)
    h2 = jnp.maximum(
        jnp.dot(h1, w1_ref[...], preferred_element_type=jnp.float32) + b1_ref[...], 0.0)
    o_ref[...] = jnp.maximum(
        jnp.dot(h2, w2_ref[...], preferred_element_type=jnp.float32) + b2_ref[...], 0.0)


def _mid_body(sp_ref, degp_ref, w3_ref, b3_ref, wd2_ref, ws2_ref, bd2_ref,
              pd2_ref, ps2_ref):
    ssum = sp_ref[0] + sp_ref[1]
    deg = degp_ref[0, :, 0:1] + degp_ref[1, :, 0:1]
    h = jnp.maximum(
        jnp.dot(ssum, w3_ref[...], preferred_element_type=jnp.float32) + deg * b3_ref[...],
        0.0)
    pd2_ref[...] = jnp.dot(h, wd2_ref[...], preferred_element_type=jnp.float32) + bd2_ref[...]
    ps2_ref[...] = jnp.dot(h, ws2_ref[...], preferred_element_type=jnp.float32)


def _final_body(sp_ref, degp_ref, w3_ref, b3_ref, o_ref):
    ssum = sp_ref[0] + sp_ref[1]
    deg = degp_ref[0, :, 0:1] + degp_ref[1, :, 0:1]
    o_ref[...] = jnp.dot(ssum, w3_ref[...], preferred_element_type=jnp.float32) + deg * b3_ref[...]


def _tables(x, wd, ws, bd):
    din = x.shape[1]
    return pl.pallas_call(
        _tables_body,
        grid=(N // TILE_N,),
        in_specs=[
            pl.BlockSpec((TILE_N, din), lambda i: (i, 0)),
            pl.BlockSpec((din, 64), lambda i: (0, 0)),
            pl.BlockSpec((din, 64), lambda i: (0, 0)),
            pl.BlockSpec((1, 64), lambda i: (0, 0)),
        ],
        out_specs=[
            pl.BlockSpec((TILE_N, 64), lambda i: (i, 0)),
            pl.BlockSpec((TILE_N, 64), lambda i: (i, 0)),
        ],
        out_shape=[
            jax.ShapeDtypeStruct((N, 64), jnp.float32),
            jax.ShapeDtypeStruct((N, 64), jnp.float32),
        ],
    )(x, wd, ws, bd)


def _mlp(g, w1, b1, w2, b2):
    return pl.pallas_call(
        _mlp_body,
        grid=(E // TILE_E,),
        in_specs=[
            pl.BlockSpec((TILE_E, 64), lambda i: (i, 0)),
            pl.BlockSpec((64, 64), lambda i: (0, 0)),
            pl.BlockSpec((1, 64), lambda i: (0, 0)),
            pl.BlockSpec((64, 64), lambda i: (0, 0)),
            pl.BlockSpec((1, 64), lambda i: (0, 0)),
        ],
        out_specs=pl.BlockSpec((TILE_E, 64), lambda i: (i, 0)),
        out_shape=jax.ShapeDtypeStruct((E, 64), jnp.float32),
    )(g, w1, b1, w2, b2)


def _mid(sp, degp, w3, b3, wd2, ws2, bd2):
    return pl.pallas_call(
        _mid_body,
        grid=(N // TILE_N,),
        in_specs=[
            pl.BlockSpec((2, TILE_N, 64), lambda i: (0, i, 0)),
            pl.BlockSpec((2, TILE_N, 16), lambda i: (0, i, 0)),
            pl.BlockSpec((64, 64), lambda i: (0, 0)),
            pl.BlockSpec((1, 64), lambda i: (0, 0)),
            pl.BlockSpec((64, 64), lambda i: (0, 0)),
            pl.BlockSpec((64, 64), lambda i: (0, 0)),
            pl.BlockSpec((1, 64), lambda i: (0, 0)),
        ],
        out_specs=[
            pl.BlockSpec((TILE_N, 64), lambda i: (i, 0)),
            pl.BlockSpec((TILE_N, 64), lambda i: (i, 0)),
        ],
        out_shape=[
            jax.ShapeDtypeStruct((N, 64), jnp.float32),
            jax.ShapeDtypeStruct((N, 64), jnp.float32),
        ],
    )(sp, degp, w3, b3, wd2, ws2, bd2)


def _final(sp, degp, w3, b3):
    return pl.pallas_call(
        _final_body,
        grid=(N // TILE_N,),
        in_specs=[
            pl.BlockSpec((2, TILE_N, 64), lambda i: (0, i, 0)),
            pl.BlockSpec((2, TILE_N, 16), lambda i: (0, i, 0)),
            pl.BlockSpec((64, 128), lambda i: (0, 0)),
            pl.BlockSpec((1, 128), lambda i: (0, 0)),
        ],
        out_specs=pl.BlockSpec((TILE_N, 128), lambda i: (i, 0)),
        out_shape=jax.ShapeDtypeStruct((N, 128), jnp.float32),
    )(sp, degp, w3, b3)


def kernel(x, edge_index, params):
    src = edge_index[0]
    dst = edge_index[1]
    (W0, b0), (W1, b1), (W2, b2), (W3, b3) = params["conv1"]
    (V0, c0), (V1, c1), (V2, c2), (V3, c3) = params["conv2"]
    W0i, W0j = W0[:128], W0[128:]
    V0i, V0j = V0[:64], V0[64:]

    zeros64 = jnp.zeros((N, 64), jnp.float32)
    zeros16 = jnp.zeros((N, 16), jnp.float32)
    onesC = jnp.ones((C, 16), jnp.float32)

    degp = _degree(dst, onesC, zeros16)

    pd1, ps1 = _tables(x, W0i - W0j, W0j, b0.reshape(1, 64))
    g1 = _gather_add(dst, src, pd1, ps1)
    h3_1 = _mlp(g1, W1, b1.reshape(1, 64), W2, b2.reshape(1, 64))
    s1p = _segsum(dst, h3_1, zeros64)

    pd2, ps2 = _mid(s1p, degp, W3, b3.reshape(1, 64),
                    V0i - V0j, V0j, c0.reshape(1, 64))
    g2 = _gather_add(dst, src, pd2, ps2)
    h3_2 = _mlp(g2, V1, c1.reshape(1, 64), V2, c2.reshape(1, 64))
    s2p = _segsum(dst, h3_2, zeros64)

    return _final(s2p, degp, V3, c3.reshape(1, 128))


# speedup vs baseline: 1.9864x; 1.5138x over previous
import functools

import jax
import jax.numpy as jnp
from jax import lax
from jax.experimental import pallas as pl
from jax.experimental.pallas import tpu as pltpu
from jax.experimental.pallas import tpu_sc as plsc

N = 10000
E = 320000
NW = 32
C = 128
NCH = 78
NEXTRA = E // C - NCH * NW
RPT = N // 16

_mesh = plsc.VectorSubcoreMesh(core_axis_name="c", subcore_axis_name="s")


C2 = C // 2


@functools.partial(
    pl.kernel,
    out_type=jax.ShapeDtypeStruct((E // 2, 128), jnp.float32),
    mesh=_mesh,
    scratch_types=[
        pltpu.VMEM((2, C), jnp.int32),
        pltpu.VMEM((2, C), jnp.int32),
        pltpu.VMEM((2, C, 64), jnp.float32),
        pltpu.VMEM((2, C, 64), jnp.float32),
        pltpu.VMEM((2, C2, 128), jnp.float32),
        pltpu.SemaphoreType.DMA,
        pltpu.SemaphoreType.DMA,
        pltpu.SemaphoreType.DMA,
        pltpu.SemaphoreType.DMA,
        pltpu.SemaphoreType.DMA,
        pltpu.SemaphoreType.DMA,
    ],
    compiler_params=pltpu.CompilerParams(use_tc_tiling_on_sc=False),
)
def _gather_add(dst_h, src_h, pd_h, ps_h, g_h, idxd, idxs, bufa, bufb, bufc,
                semi0, semi1, semg0, semg1, semo0, semo1):
    c = lax.axis_index("c")
    s = lax.axis_index("s")
    w = c * 16 + s
    tbase = w * (NCH * C)
    semi = [semi0, semi1]
    semg = [semg0, semg1]
    semo = [semo0, semo1]

    def issue_idx(k, b):
        base = tbase + k * C
        pltpu.async_copy(dst_h.at[pl.ds(base, C)], idxd.at[b], semi[b])
        pltpu.async_copy(src_h.at[pl.ds(base, C)], idxs.at[b], semi[b])

    def wait_idx(b):
        pltpu.make_async_copy(dst_h.at[pl.ds(0, C)], idxd.at[b], semi[b]).wait()
        pltpu.make_async_copy(src_h.at[pl.ds(0, C)], idxs.at[b], semi[b]).wait()

    def issue_gather(b):
        pltpu.async_copy(pd_h.at[idxd.at[b]], bufa.at[b], semg[b])
        pltpu.async_copy(ps_h.at[idxs.at[b]], bufb.at[b], semg[b])

    def wait_gather(b):
        pltpu.make_async_copy(pd_h.at[idxd.at[b]], bufa.at[b], semg[b]).wait()
        pltpu.make_async_copy(ps_h.at[idxs.at[b]], bufb.at[b], semg[b]).wait()

    tbase2 = tbase // 2

    def issue_store(k, b):
        pltpu.async_copy(bufc.at[b], g_h.at[pl.ds(tbase2 + k * C2, C2)], semo[b])

    def wait_store(b):
        pltpu.make_async_copy(bufc.at[b], g_h.at[pl.ds(tbase2, C2)], semo[b]).wait()

    def add_rows(b, nrows):
        def row(r, carry):
            for p in range(2):
                for j in range(4):
                    si = pl.ds(j * 16, 16)
                    so = pl.ds(p * 64 + j * 16, 16)
                    bufc[b, r, so] = bufa[b, 2 * r + p, si] + bufb[b, 2 * r + p, si]
            return carry
        lax.fori_loop(0, nrows // 2, row, 0, unroll=4)

    issue_idx(0, 0)
    issue_idx(1, 1)
    wait_idx(0)
    issue_gather(0)
    wait_gather(0)
    issue_idx(2, 0)
    wait_idx(1)
    issue_gather(1)
    add_rows(0, C)
    issue_store(0, 0)
    wait_gather(1)
    issue_idx(3, 1)
    wait_idx(0)
    wait_store(0)
    issue_gather(0)
    add_rows(1, C)
    issue_store(1, 1)

    def pair(kk, carry):
        for b in (0, 1):
            k = 2 + 2 * kk + b
            o = 1 - b
            wait_gather(b)
            issue_idx(k + 2, b)
            wait_idx(o)
            wait_store(o)
            issue_gather(o)
            add_rows(b, C)
            issue_store(k, b)
        return carry

    lax.fori_loop(0, (NCH - 4) // 2, pair, 0)
    wait_gather(0)
    wait_idx(1)
    wait_store(1)
    issue_gather(1)
    add_rows(0, C)
    issue_store(NCH - 2, 0)
    wait_gather(1)
    add_rows(1, C)
    issue_store(NCH - 1, 1)
    wait_store(0)
    wait_store(1)

    @pl.when(w < NEXTRA)
    def _():
        base = (NCH * NW + w) * C
        pltpu.sync_copy(dst_h.at[pl.ds(base, C)], idxd.at[0])
        pltpu.sync_copy(src_h.at[pl.ds(base, C)], idxs.at[0])
        cpa = pltpu.async_copy(pd_h.at[idxd.at[0]], bufa.at[0], semg[0])
        cpb = pltpu.async_copy(ps_h.at[idxs.at[0]], bufb.at[0], semg[0])
        cpa.wait()
        cpb.wait()
        add_rows(0, C)
        pltpu.sync_copy(bufc.at[0], g_h.at[pl.ds(base // 2, C2)])


@functools.partial(
    pl.kernel,
    out_type=jax.ShapeDtypeStruct((2, N, 64), jnp.float32),
    mesh=_mesh,
    scratch_types=[
        pltpu.VMEM((2, C2), jnp.int32),
        pltpu.VMEM((2, C2), jnp.int32),
        pltpu.VMEM((2, C2, 64), jnp.float32),
        pltpu.VMEM((2, C2, 64), jnp.float32),
        pltpu.VMEM_SHARED((N, 64), jnp.float32),
        pltpu.SemaphoreType.DMA,
        pltpu.SemaphoreType.DMA,
        pltpu.SemaphoreType.DMA,
        pltpu.SemaphoreType.DMA,
    ],
    compiler_params=pltpu.CompilerParams(use_tc_tiling_on_sc=False),
)
def _segsum(dste_h, dsto_h, val_h, zeros_h, out_h, idxe, idxo, bufe, bufo, accum,
            seml0, seml1, semc0, semc1):
    c = lax.axis_index("c")
    s = lax.axis_index("s")
    rsl = pl.ds(s * RPT, RPT)
    pltpu.sync_copy(zeros_h.at[rsl], accum.at[rsl])
    plsc.subcore_barrier()
    w = c * 16 + s
    tbase2 = w * (NCH * C2)
    seml = [seml0, seml1]
    semc = [semc0, semc1]

    def issue_load(k, b):
        base = tbase2 + k * C2
        pltpu.async_copy(dste_h.at[pl.ds(base, C2)], idxe.at[b], seml[b])
        pltpu.async_copy(dsto_h.at[pl.ds(base, C2)], idxo.at[b], seml[b])
        pltpu.async_copy(val_h.at[pl.ds(base, C2), pl.ds(0, 64)], bufe.at[b], seml[b])
        pltpu.async_copy(val_h.at[pl.ds(base, C2), pl.ds(64, 64)], bufo.at[b], seml[b])

    def wait_load(b):
        pltpu.make_async_copy(dste_h.at[pl.ds(0, C2)], idxe.at[b], seml[b]).wait()
        pltpu.make_async_copy(dsto_h.at[pl.ds(0, C2)], idxo.at[b], seml[b]).wait()
        pltpu.make_async_copy(val_h.at[pl.ds(0, C2), pl.ds(0, 64)], bufe.at[b], seml[b]).wait()
        pltpu.make_async_copy(val_h.at[pl.ds(0, C2), pl.ds(64, 64)], bufo.at[b], seml[b]).wait()

    def issue_scat(b):
        pltpu.async_copy(bufe.at[b], accum.at[idxe.at[b]], semc[b], add=True)
        pltpu.async_copy(bufo.at[b], accum.at[idxo.at[b]], semc[b], add=True)

    def wait_scat(b):
        pltpu.make_async_copy(bufe.at[b], accum.at[idxe.at[b]], semc[b]).wait()
        pltpu.make_async_copy(bufo.at[b], accum.at[idxo.at[b]], semc[b]).wait()

    issue_load(0, 0)
    wait_load(0)
    issue_scat(0)
    issue_load(1, 1)

    def pair(kk, carry):
        for b in (1, 0):
            k = 1 + 2 * kk + (1 - b)
            o = 1 - b
            wait_load(b)
            issue_scat(b)
            wait_scat(o)
            issue_load(k + 1, o)
        return carry

    lax.fori_loop(0, (NCH - 2) // 2, pair, 0)
    wait_load(1)
    issue_scat(1)
    wait_scat(0)
    wait_scat(1)

    @pl.when(w < NEXTRA)
    def _():
        base = (NCH * NW + w) * C2
        pltpu.sync_copy(dste_h.at[pl.ds(base, C2)], idxe.at[0])
        pltpu.sync_copy(dsto_h.at[pl.ds(base, C2)], idxo.at[0])
        pltpu.sync_copy(val_h.at[pl.ds(base, C2), pl.ds(0, 64)], bufe.at[0])
        pltpu.sync_copy(val_h.at[pl.ds(base, C2), pl.ds(64, 64)], bufo.at[0])
        pltpu.sync_copy(bufe.at[0], accum.at[idxe.at[0]], add=True)
        pltpu.sync_copy(bufo.at[0], accum.at[idxo.at[0]], add=True)

    plsc.subcore_barrier()
    pltpu.sync_copy(accum.at[rsl], out_h.at[c, rsl])


@functools.partial(
    pl.kernel,
    out_type=jax.ShapeDtypeStruct((2, N, 16), jnp.float32),
    mesh=_mesh,
    scratch_types=[
        pltpu.VMEM((2, C), jnp.int32),
        pltpu.VMEM((C, 16), jnp.float32),
        pltpu.VMEM_SHARED((N, 16), jnp.float32),
        pltpu.SemaphoreType.DMA,
        pltpu.SemaphoreType.DMA,
        pltpu.SemaphoreType.DMA,
        pltpu.SemaphoreType.DMA,
    ],
    compiler_params=pltpu.CompilerParams(use_tc_tiling_on_sc=False),
)
def _degree(dst_h, ones_h, zeros_h, out_h, idx, buf, accum,
            seml0, seml1, semc0, semc1):
    c = lax.axis_index("c")
    s = lax.axis_index("s")
    rsl = pl.ds(s * RPT, RPT)
    pltpu.sync_copy(zeros_h.at[rsl], accum.at[rsl])
    pltpu.sync_copy(ones_h, buf)
    plsc.subcore_barrier()
    w = c * 16 + s
    tbase = w * (NCH * C)
    seml = [seml0, seml1]
    semc = [semc0, semc1]

    def issue_load(k, b):
        pltpu.async_copy(dst_h.at[pl.ds(tbase + k * C, C)], idx.at[b], seml[b])

    def wait_load(b):
        pltpu.make_async_copy(dst_h.at[pl.ds(0, C)], idx.at[b], seml[b]).wait()

    def issue_scat(b):
        pltpu.async_copy(buf, accum.at[idx.at[b]], semc[b], add=True)

    def wait_scat(b):
        pltpu.make_async_copy(buf, accum.at[idx.at[b]], semc[b]).wait()

    issue_load(0, 0)
    wait_load(0)
    issue_scat(0)
    issue_load(1, 1)

    def pair(kk, carry):
        for b in (1, 0):
            k = 1 + 2 * kk + (1 - b)
            o = 1 - b
            wait_load(b)
            issue_scat(b)
            wait_scat(o)
            issue_load(k + 1, o)
        return carry

    lax.fori_loop(0, (NCH - 2) // 2, pair, 0)
    wait_load(1)
    issue_scat(1)
    wait_scat(0)
    wait_scat(1)

    @pl.when(w < NEXTRA)
    def _():
        base = (NCH * NW + w) * C
        pltpu.sync_copy(dst_h.at[pl.ds(base, C)], idx.at[0])
        pltpu.sync_copy(buf, accum.at[idx.at[0]], add=True)

    plsc.subcore_barrier()
    pltpu.sync_copy(accum.at[rsl], out_h.at[c, rsl])


TILE_N = 2000
TILE_E = 4000


def _tables_body(x_ref, wd_ref, ws_ref, bd_ref, pd_ref, ps_ref):
    xb = x_ref[...]
    pd_ref[...] = jnp.dot(xb, wd_ref[...], preferred_element_type=jnp.float32) + bd_ref[...]
    ps_ref[...] = jnp.dot(xb, ws_ref[...], preferred_element_type=jnp.float32)


def _mlp_body(g_ref, w1_ref, b1_ref, w2_ref, b2_ref, o_ref):
    gb = g_ref[...]
    h1 = jnp.maximum(jnp.concatenate([gb[:, :64], gb[:, 64:]], axis=0), 0.0)
    h2 = jnp.maximum(
        jnp.dot(h1, w1_ref[...], preferred_element_type=jnp.float32) + b1_ref[...], 0.0)
    h3 = jnp.maximum(
        jnp.dot(h2, w2_ref[...], preferred_element_type=jnp.float32) + b2_ref[...], 0.0)
    half = h3.shape[0] // 2
    o_ref[...] = jnp.concatenate([h3[:half], h3[half:]], axis=1)


def _mid_body(sp_ref, degp_ref, w3_ref, b3_ref, wd2_ref, ws2_ref, bd2_ref,
              pd2_ref, ps2_ref):
    ssum = sp_ref[0] + sp_ref[1]
    deg = degp_ref[0, :, 0:1] + degp_ref[1, :, 0:1]
    h = jnp.maximum(
        jnp.dot(ssum, w3_ref[...], preferred_element_type=jnp.float32) + deg * b3_ref[...],
        0.0)
    pd2_ref[...] = jnp.dot(h, wd2_ref[...], preferred_element_type=jnp.float32) + bd2_ref[...]
    ps2_ref[...] = jnp.dot(h, ws2_ref[...], preferred_element_type=jnp.float32)


def _final_body(sp_ref, degp_ref, w3_ref, b3_ref, o_ref):
    ssum = sp_ref[0] + sp_ref[1]
    deg = degp_ref[0, :, 0:1] + degp_ref[1, :, 0:1]
    o_ref[...] = jnp.dot(ssum, w3_ref[...], preferred_element_type=jnp.float32) + deg * b3_ref[...]


def _tables(x, wd, ws, bd):
    din = x.shape[1]
    return pl.pallas_call(
        _tables_body,
        grid=(N // TILE_N,),
        in_specs=[
            pl.BlockSpec((TILE_N, din), lambda i: (i, 0)),
            pl.BlockSpec((din, 64), lambda i: (0, 0)),
            pl.BlockSpec((din, 64), lambda i: (0, 0)),
            pl.BlockSpec((1, 64), lambda i: (0, 0)),
        ],
        out_specs=[
            pl.BlockSpec((TILE_N, 64), lambda i: (i, 0)),
            pl.BlockSpec((TILE_N, 64), lambda i: (i, 0)),
        ],
        out_shape=[
            jax.ShapeDtypeStruct((N, 64), jnp.float32),
            jax.ShapeDtypeStruct((N, 64), jnp.float32),
        ],
    )(x, wd, ws, bd)


def _mlp(g, w1, b1, w2, b2):
    return pl.pallas_call(
        _mlp_body,
        grid=(E // TILE_E,),
        in_specs=[
            pl.BlockSpec((TILE_E // 2, 128), lambda i: (i, 0)),
            pl.BlockSpec((64, 64), lambda i: (0, 0)),
            pl.BlockSpec((1, 64), lambda i: (0, 0)),
            pl.BlockSpec((64, 64), lambda i: (0, 0)),
            pl.BlockSpec((1, 64), lambda i: (0, 0)),
        ],
        out_specs=pl.BlockSpec((TILE_E // 2, 128), lambda i: (i, 0)),
        out_shape=jax.ShapeDtypeStruct((E // 2, 128), jnp.float32),
    )(g, w1, b1, w2, b2)


def _mid(sp, degp, w3, b3, wd2, ws2, bd2):
    return pl.pallas_call(
        _mid_body,
        grid=(N // TILE_N,),
        in_specs=[
            pl.BlockSpec((2, TILE_N, 64), lambda i: (0, i, 0)),
            pl.BlockSpec((2, TILE_N, 16), lambda i: (0, i, 0)),
            pl.BlockSpec((64, 64), lambda i: (0, 0)),
            pl.BlockSpec((1, 64), lambda i: (0, 0)),
            pl.BlockSpec((64, 64), lambda i: (0, 0)),
            pl.BlockSpec((64, 64), lambda i: (0, 0)),
            pl.BlockSpec((1, 64), lambda i: (0, 0)),
        ],
        out_specs=[
            pl.BlockSpec((TILE_N, 64), lambda i: (i, 0)),
            pl.BlockSpec((TILE_N, 64), lambda i: (i, 0)),
        ],
        out_shape=[
            jax.ShapeDtypeStruct((N, 64), jnp.float32),
            jax.ShapeDtypeStruct((N, 64), jnp.float32),
        ],
    )(sp, degp, w3, b3, wd2, ws2, bd2)


def _final(sp, degp, w3, b3):
    return pl.pallas_call(
        _final_body,
        grid=(N // TILE_N,),
        in_specs=[
            pl.BlockSpec((2, TILE_N, 64), lambda i: (0, i, 0)),
            pl.BlockSpec((2, TILE_N, 16), lambda i: (0, i, 0)),
            pl.BlockSpec((64, 128), lambda i: (0, 0)),
            pl.BlockSpec((1, 128), lambda i: (0, 0)),
        ],
        out_specs=pl.BlockSpec((TILE_N, 128), lambda i: (i, 0)),
        out_shape=jax.ShapeDtypeStruct((N, 128), jnp.float32),
    )(sp, degp, w3, b3)


def kernel(x, edge_index, params):
    src = edge_index[0]
    dst = edge_index[1]
    (W0, b0), (W1, b1), (W2, b2), (W3, b3) = params["conv1"]
    (V0, c0), (V1, c1), (V2, c2), (V3, c3) = params["conv2"]
    W0i, W0j = W0[:128], W0[128:]
    V0i, V0j = V0[:64], V0[64:]

    zeros64 = jnp.zeros((N, 64), jnp.float32)
    zeros16 = jnp.zeros((N, 16), jnp.float32)
    onesC = jnp.ones((C, 16), jnp.float32)

    dste = dst[0::2]
    dsto = dst[1::2]

    degp = _degree(dst, onesC, zeros16)

    pd1, ps1 = _tables(x, W0i - W0j, W0j, b0.reshape(1, 64))
    g1 = _gather_add(dst, src, pd1, ps1)
    h3_1 = _mlp(g1, W1, b1.reshape(1, 64), W2, b2.reshape(1, 64))
    s1p = _segsum(dste, dsto, h3_1, zeros64)

    pd2, ps2 = _mid(s1p, degp, W3, b3.reshape(1, 64),
                    V0i - V0j, V0j, c0.reshape(1, 64))
    g2 = _gather_add(dst, src, pd2, ps2)
    h3_2 = _mlp(g2, V1, c1.reshape(1, 64), V2, c2.reshape(1, 64))
    s2p = _segsum(dste, dsto, h3_2, zeros64)

    return _final(s2p, degp, V3, c3.reshape(1, 128))


# speedup vs baseline: 2.6177x; 1.3178x over previous
import functools

import jax
import jax.numpy as jnp
from jax import lax
from jax.experimental import pallas as pl
from jax.experimental.pallas import tpu as pltpu
from jax.experimental.pallas import tpu_sc as plsc

N = 10000
E = 320000
NW = 32
C = 128
NCH = 78
NEXTRA = E // C - NCH * NW
RPT = N // 16

_mesh = plsc.VectorSubcoreMesh(core_axis_name="c", subcore_axis_name="s")


C2 = C // 2


@functools.partial(
    pl.kernel,
    out_type=jax.ShapeDtypeStruct((E // 2, 128), jnp.float32),
    mesh=_mesh,
    scratch_types=[
        pltpu.VMEM((2, C), jnp.int32),
        pltpu.VMEM((2, C), jnp.int32),
        pltpu.VMEM((2, C, 64), jnp.float32),
        pltpu.VMEM((2, C, 64), jnp.float32),
        pltpu.VMEM((2, C2, 128), jnp.float32),
        pltpu.SemaphoreType.DMA,
        pltpu.SemaphoreType.DMA,
        pltpu.SemaphoreType.DMA,
        pltpu.SemaphoreType.DMA,
        pltpu.SemaphoreType.DMA,
        pltpu.SemaphoreType.DMA,
    ],
    compiler_params=pltpu.CompilerParams(use_tc_tiling_on_sc=False),
)
def _gather_add(dst_h, src_h, pd_h, ps_h, g_h, idxd, idxs, bufa, bufb, bufc,
                semi0, semi1, semg0, semg1, semo0, semo1):
    c = lax.axis_index("c")
    s = lax.axis_index("s")
    w = c * 16 + s
    tbase = w * (NCH * C)
    semi = [semi0, semi1]
    semg = [semg0, semg1]
    semo = [semo0, semo1]

    def issue_idx(k, b):
        base = tbase + k * C
        pltpu.async_copy(dst_h.at[pl.ds(base, C)], idxd.at[b], semi[b])
        pltpu.async_copy(src_h.at[pl.ds(base, C)], idxs.at[b], semi[b])

    def wait_idx(b):
        pltpu.make_async_copy(dst_h.at[pl.ds(0, C)], idxd.at[b], semi[b]).wait()
        pltpu.make_async_copy(src_h.at[pl.ds(0, C)], idxs.at[b], semi[b]).wait()

    def issue_gather(b):
        pltpu.async_copy(pd_h.at[idxd.at[b]], bufa.at[b], semg[b])
        pltpu.async_copy(ps_h.at[idxs.at[b]], bufb.at[b], semg[b])

    def wait_gather(b):
        pltpu.make_async_copy(pd_h.at[idxd.at[b]], bufa.at[b], semg[b]).wait()
        pltpu.make_async_copy(ps_h.at[idxs.at[b]], bufb.at[b], semg[b]).wait()

    tbase2 = tbase // 2

    def issue_store(k, b):
        pltpu.async_copy(bufc.at[b], g_h.at[pl.ds(tbase2 + k * C2, C2)], semo[b])

    def wait_store(b):
        pltpu.make_async_copy(bufc.at[b], g_h.at[pl.ds(tbase2, C2)], semo[b]).wait()

    def add_rows(b, nrows):
        @plsc.parallel_loop(0, nrows // 2, unroll=4)
        def row(r):
            for p in range(2):
                for j in range(4):
                    si = pl.ds(j * 16, 16)
                    so = pl.ds(p * 64 + j * 16, 16)
                    bufc[b, r, so] = bufa[b, 2 * r + p, si] + bufb[b, 2 * r + p, si]

    issue_idx(0, 0)
    issue_idx(1, 1)
    wait_idx(0)
    issue_gather(0)
    wait_gather(0)
    issue_idx(2, 0)
    wait_idx(1)
    issue_gather(1)
    add_rows(0, C)
    issue_store(0, 0)
    wait_gather(1)
    issue_idx(3, 1)
    wait_idx(0)
    wait_store(0)
    issue_gather(0)
    add_rows(1, C)
    issue_store(1, 1)

    def pair(kk, carry):
        for b in (0, 1):
            k = 2 + 2 * kk + b
            o = 1 - b
            wait_gather(b)
            issue_idx(k + 2, b)
            wait_idx(o)
            wait_store(o)
            issue_gather(o)
            add_rows(b, C)
            issue_store(k, b)
        return carry

    lax.fori_loop(0, (NCH - 4) // 2, pair, 0)
    wait_gather(0)
    wait_idx(1)
    wait_store(1)
    issue_gather(1)
    add_rows(0, C)
    issue_store(NCH - 2, 0)
    wait_gather(1)
    add_rows(1, C)
    issue_store(NCH - 1, 1)
    wait_store(0)
    wait_store(1)

    @pl.when(w < NEXTRA)
    def _():
        base = (NCH * NW + w) * C
        pltpu.sync_copy(dst_h.at[pl.ds(base, C)], idxd.at[0])
        pltpu.sync_copy(src_h.at[pl.ds(base, C)], idxs.at[0])
        cpa = pltpu.async_copy(pd_h.at[idxd.at[0]], bufa.at[0], semg[0])
        cpb = pltpu.async_copy(ps_h.at[idxs.at[0]], bufb.at[0], semg[0])
        cpa.wait()
        cpb.wait()
        add_rows(0, C)
        pltpu.sync_copy(bufc.at[0], g_h.at[pl.ds(base // 2, C2)])


@functools.partial(
    pl.kernel,
    out_type=jax.ShapeDtypeStruct((2, N, 64), jnp.float32),
    mesh=_mesh,
    scratch_types=[
        pltpu.VMEM((2, C2), jnp.int32),
        pltpu.VMEM((2, C2), jnp.int32),
        pltpu.VMEM((2, C2, 64), jnp.float32),
        pltpu.VMEM((2, C2, 64), jnp.float32),
        pltpu.VMEM_SHARED((N, 64), jnp.float32),
        pltpu.SemaphoreType.DMA,
        pltpu.SemaphoreType.DMA,
        pltpu.SemaphoreType.DMA,
        pltpu.SemaphoreType.DMA,
    ],
    compiler_params=pltpu.CompilerParams(use_tc_tiling_on_sc=False),
)
def _segsum(dste_h, dsto_h, val_h, zeros_h, out_h, idxe, idxo, bufe, bufo, accum,
            seml0, seml1, semc0, semc1):
    c = lax.axis_index("c")
    s = lax.axis_index("s")
    rsl = pl.ds(s * RPT, RPT)
    pltpu.sync_copy(zeros_h.at[rsl], accum.at[rsl])
    plsc.subcore_barrier()
    w = c * 16 + s
    tbase2 = w * (NCH * C2)
    seml = [seml0, seml1]
    semc = [semc0, semc1]

    def issue_load(k, b):
        base = tbase2 + k * C2
        pltpu.async_copy(dste_h.at[pl.ds(base, C2)], idxe.at[b], seml[b])
        pltpu.async_copy(dsto_h.at[pl.ds(base, C2)], idxo.at[b], seml[b])
        pltpu.async_copy(val_h.at[pl.ds(base, C2), pl.ds(0, 64)], bufe.at[b], seml[b])
        pltpu.async_copy(val_h.at[pl.ds(base, C2), pl.ds(64, 64)], bufo.at[b], seml[b])

    def wait_load(b):
        pltpu.make_async_copy(dste_h.at[pl.ds(0, C2)], idxe.at[b], seml[b]).wait()
        pltpu.make_async_copy(dsto_h.at[pl.ds(0, C2)], idxo.at[b], seml[b]).wait()
        pltpu.make_async_copy(val_h.at[pl.ds(0, C2), pl.ds(0, 64)], bufe.at[b], seml[b]).wait()
        pltpu.make_async_copy(val_h.at[pl.ds(0, C2), pl.ds(64, 64)], bufo.at[b], seml[b]).wait()

    def issue_scat(b):
        pltpu.async_copy(bufe.at[b], accum.at[idxe.at[b]], semc[b], add=True)
        pltpu.async_copy(bufo.at[b], accum.at[idxo.at[b]], semc[b], add=True)

    def wait_scat(b):
        pltpu.make_async_copy(bufe.at[b], accum.at[idxe.at[b]], semc[b]).wait()
        pltpu.make_async_copy(bufo.at[b], accum.at[idxo.at[b]], semc[b]).wait()

    issue_load(0, 0)
    wait_load(0)
    issue_scat(0)
    issue_load(1, 1)

    def pair(kk, carry):
        for b in (1, 0):
            k = 1 + 2 * kk + (1 - b)
            o = 1 - b
            wait_load(b)
            issue_scat(b)
            wait_scat(o)
            issue_load(k + 1, o)
        return carry

    lax.fori_loop(0, (NCH - 2) // 2, pair, 0)
    wait_load(1)
    issue_scat(1)
    wait_scat(0)
    wait_scat(1)

    @pl.when(w < NEXTRA)
    def _():
        base = (NCH * NW + w) * C2
        pltpu.sync_copy(dste_h.at[pl.ds(base, C2)], idxe.at[0])
        pltpu.sync_copy(dsto_h.at[pl.ds(base, C2)], idxo.at[0])
        pltpu.sync_copy(val_h.at[pl.ds(base, C2), pl.ds(0, 64)], bufe.at[0])
        pltpu.sync_copy(val_h.at[pl.ds(base, C2), pl.ds(64, 64)], bufo.at[0])
        pltpu.sync_copy(bufe.at[0], accum.at[idxe.at[0]], add=True)
        pltpu.sync_copy(bufo.at[0], accum.at[idxo.at[0]], add=True)

    plsc.subcore_barrier()
    pltpu.sync_copy(accum.at[rsl], out_h.at[c, rsl])


@functools.partial(
    pl.kernel,
    out_type=jax.ShapeDtypeStruct((2, N, 16), jnp.float32),
    mesh=_mesh,
    scratch_types=[
        pltpu.VMEM((2, C), jnp.int32),
        pltpu.VMEM((C, 16), jnp.float32),
        pltpu.VMEM_SHARED((N, 16), jnp.float32),
        pltpu.SemaphoreType.DMA,
        pltpu.SemaphoreType.DMA,
        pltpu.SemaphoreType.DMA,
        pltpu.SemaphoreType.DMA,
    ],
    compiler_params=pltpu.CompilerParams(use_tc_tiling_on_sc=False),
)
def _degree(dst_h, ones_h, zeros_h, out_h, idx, buf, accum,
            seml0, seml1, semc0, semc1):
    c = lax.axis_index("c")
    s = lax.axis_index("s")
    rsl = pl.ds(s * RPT, RPT)
    pltpu.sync_copy(zeros_h.at[rsl], accum.at[rsl])
    pltpu.sync_copy(ones_h, buf)
    plsc.subcore_barrier()
    w = c * 16 + s
    tbase = w * (NCH * C)
    seml = [seml0, seml1]
    semc = [semc0, semc1]

    def issue_load(k, b):
        pltpu.async_copy(dst_h.at[pl.ds(tbase + k * C, C)], idx.at[b], seml[b])

    def wait_load(b):
        pltpu.make_async_copy(dst_h.at[pl.ds(0, C)], idx.at[b], seml[b]).wait()

    def issue_scat(b):
        pltpu.async_copy(buf, accum.at[idx.at[b]], semc[b], add=True)

    def wait_scat(b):
        pltpu.make_async_copy(buf, accum.at[idx.at[b]], semc[b]).wait()

    issue_load(0, 0)
    wait_load(0)
    issue_scat(0)
    issue_load(1, 1)

    def pair(kk, carry):
        for b in (1, 0):
            k = 1 + 2 * kk + (1 - b)
            o = 1 - b
            wait_load(b)
            issue_scat(b)
            wait_scat(o)
            issue_load(k + 1, o)
        return carry

    lax.fori_loop(0, (NCH - 2) // 2, pair, 0)
    wait_load(1)
    issue_scat(1)
    wait_scat(0)
    wait_scat(1)

    @pl.when(w < NEXTRA)
    def _():
        base = (NCH * NW + w) * C
        pltpu.sync_copy(dst_h.at[pl.ds(base, C)], idx.at[0])
        pltpu.sync_copy(buf, accum.at[idx.at[0]], add=True)

    plsc.subcore_barrier()
    pltpu.sync_copy(accum.at[rsl], out_h.at[c, rsl])


TILE_N = 2000
TILE_E = 4000


def _tables_body(x_ref, wd_ref, ws_ref, bd_ref, pd_ref, ps_ref):
    xb = x_ref[...]
    pd_ref[...] = jnp.dot(xb, wd_ref[...], preferred_element_type=jnp.float32) + bd_ref[...]
    ps_ref[...] = jnp.dot(xb, ws_ref[...], preferred_element_type=jnp.float32)


def _mlp_body(g_ref, w1_ref, b1_ref, w2_ref, b2_ref, o_ref):
    gb = g_ref[...]
    h1 = jnp.maximum(jnp.concatenate([gb[:, :64], gb[:, 64:]], axis=0), 0.0)
    h2 = jnp.maximum(
        jnp.dot(h1, w1_ref[...], preferred_element_type=jnp.float32) + b1_ref[...], 0.0)
    h3 = jnp.maximum(
        jnp.dot(h2, w2_ref[...], preferred_element_type=jnp.float32) + b2_ref[...], 0.0)
    half = h3.shape[0] // 2
    o_ref[...] = jnp.concatenate([h3[:half], h3[half:]], axis=1)


def _mid_body(sp_ref, degp_ref, w3_ref, b3_ref, wd2_ref, ws2_ref, bd2_ref,
              pd2_ref, ps2_ref):
    ssum = sp_ref[0] + sp_ref[1]
    deg = degp_ref[0, :, 0:1] + degp_ref[1, :, 0:1]
    h = jnp.maximum(
        jnp.dot(ssum, w3_ref[...], preferred_element_type=jnp.float32) + deg * b3_ref[...],
        0.0)
    pd2_ref[...] = jnp.dot(h, wd2_ref[...], preferred_element_type=jnp.float32) + bd2_ref[...]
    ps2_ref[...] = jnp.dot(h, ws2_ref[...], preferred_element_type=jnp.float32)


def _final_body(sp_ref, degp_ref, w3_ref, b3_ref, o_ref):
    ssum = sp_ref[0] + sp_ref[1]
    deg = degp_ref[0, :, 0:1] + degp_ref[1, :, 0:1]
    o_ref[...] = jnp.dot(ssum, w3_ref[...], preferred_element_type=jnp.float32) + deg * b3_ref[...]


def _tables(x, wd, ws, bd):
    din = x.shape[1]
    return pl.pallas_call(
        _tables_body,
        grid=(N // TILE_N,),
        in_specs=[
            pl.BlockSpec((TILE_N, din), lambda i: (i, 0)),
            pl.BlockSpec((din, 64), lambda i: (0, 0)),
            pl.BlockSpec((din, 64), lambda i: (0, 0)),
            pl.BlockSpec((1, 64), lambda i: (0, 0)),
        ],
        out_specs=[
            pl.BlockSpec((TILE_N, 64), lambda i: (i, 0)),
            pl.BlockSpec((TILE_N, 64), lambda i: (i, 0)),
        ],
        out_shape=[
            jax.ShapeDtypeStruct((N, 64), jnp.float32),
            jax.ShapeDtypeStruct((N, 64), jnp.float32),
        ],
    )(x, wd, ws, bd)


def _mlp(g, w1, b1, w2, b2):
    return pl.pallas_call(
        _mlp_body,
        grid=(E // TILE_E,),
        in_specs=[
            pl.BlockSpec((TILE_E // 2, 128), lambda i: (i, 0)),
            pl.BlockSpec((64, 64), lambda i: (0, 0)),
            pl.BlockSpec((1, 64), lambda i: (0, 0)),
            pl.BlockSpec((64, 64), lambda i: (0, 0)),
            pl.BlockSpec((1, 64), lambda i: (0, 0)),
        ],
        out_specs=pl.BlockSpec((TILE_E // 2, 128), lambda i: (i, 0)),
        out_shape=jax.ShapeDtypeStruct((E // 2, 128), jnp.float32),
    )(g, w1, b1, w2, b2)


def _mid(sp, degp, w3, b3, wd2, ws2, bd2):
    return pl.pallas_call(
        _mid_body,
        grid=(N // TILE_N,),
        in_specs=[
            pl.BlockSpec((2, TILE_N, 64), lambda i: (0, i, 0)),
            pl.BlockSpec((2, TILE_N, 16), lambda i: (0, i, 0)),
            pl.BlockSpec((64, 64), lambda i: (0, 0)),
            pl.BlockSpec((1, 64), lambda i: (0, 0)),
            pl.BlockSpec((64, 64), lambda i: (0, 0)),
            pl.BlockSpec((64, 64), lambda i: (0, 0)),
            pl.BlockSpec((1, 64), lambda i: (0, 0)),
        ],
        out_specs=[
            pl.BlockSpec((TILE_N, 64), lambda i: (i, 0)),
            pl.BlockSpec((TILE_N, 64), lambda i: (i, 0)),
        ],
        out_shape=[
            jax.ShapeDtypeStruct((N, 64), jnp.float32),
            jax.ShapeDtypeStruct((N, 64), jnp.float32),
        ],
    )(sp, degp, w3, b3, wd2, ws2, bd2)


def _final(sp, degp, w3, b3):
    return pl.pallas_call(
        _final_body,
        grid=(N // TILE_N,),
        in_specs=[
            pl.BlockSpec((2, TILE_N, 64), lambda i: (0, i, 0)),
            pl.BlockSpec((2, TILE_N, 16), lambda i: (0, i, 0)),
            pl.BlockSpec((64, 128), lambda i: (0, 0)),
            pl.BlockSpec((1, 128), lambda i: (0, 0)),
        ],
        out_specs=pl.BlockSpec((TILE_N, 128), lambda i: (i, 0)),
        out_shape=jax.ShapeDtypeStruct((N, 128), jnp.float32),
    )(sp, degp, w3, b3)


def kernel(x, edge_index, params):
    src = edge_index[0]
    dst = edge_index[1]
    (W0, b0), (W1, b1), (W2, b2), (W3, b3) = params["conv1"]
    (V0, c0), (V1, c1), (V2, c2), (V3, c3) = params["conv2"]
    W0i, W0j = W0[:128], W0[128:]
    V0i, V0j = V0[:64], V0[64:]

    zeros64 = jnp.zeros((N, 64), jnp.float32)
    zeros16 = jnp.zeros((N, 16), jnp.float32)
    onesC = jnp.ones((C, 16), jnp.float32)

    dste = dst[0::2]
    dsto = dst[1::2]

    degp = _degree(dst, onesC, zeros16)

    pd1, ps1 = _tables(x, W0i - W0j, W0j, b0.reshape(1, 64))
    g1 = _gather_add(dst, src, pd1, ps1)
    h3_1 = _mlp(g1, W1, b1.reshape(1, 64), W2, b2.reshape(1, 64))
    s1p = _segsum(dste, dsto, h3_1, zeros64)

    pd2, ps2 = _mid(s1p, degp, W3, b3.reshape(1, 64),
                    V0i - V0j, V0j, c0.reshape(1, 64))
    g2 = _gather_add(dst, src, pd2, ps2)
    h3_2 = _mlp(g2, V1, c1.reshape(1, 64), V2, c2.reshape(1, 64))
    s2p = _segsum(dste, dsto, h3_2, zeros64)

    return _final(s2p, degp, V3, c3.reshape(1, 128))


# speedup vs baseline: 2.8735x; 1.0977x over previous
import functools

import jax
import jax.numpy as jnp
from jax import lax
from jax.experimental import pallas as pl
from jax.experimental.pallas import tpu as pltpu
from jax.experimental.pallas import tpu_sc as plsc

N = 10000
E = 320000
NW = 32
C = 128
C2 = C // 2
RPT = N // 16

EA = C * NW * 38
EB = E - EA

_mesh = plsc.VectorSubcoreMesh(core_axis_name="c", subcore_axis_name="s")
_sc_params = pltpu.CompilerParams(use_tc_tiling_on_sc=False)


def _make_gather(ne, nch, nextra):
    @functools.partial(
        pl.kernel,
        out_type=jax.ShapeDtypeStruct((ne // 2, 128), jnp.float32),
        mesh=_mesh,
        scratch_types=[
            pltpu.VMEM((2, C), jnp.int32),
            pltpu.VMEM((2, C), jnp.int32),
            pltpu.VMEM((2, C, 64), jnp.float32),
            pltpu.VMEM((2, C, 64), jnp.float32),
            pltpu.VMEM((2, C2, 128), jnp.float32),
            pltpu.SemaphoreType.DMA,
            pltpu.SemaphoreType.DMA,
            pltpu.SemaphoreType.DMA,
            pltpu.SemaphoreType.DMA,
            pltpu.SemaphoreType.DMA,
            pltpu.SemaphoreType.DMA,
        ],
        compiler_params=_sc_params,
    )
    def gather_add(dst_h, src_h, pd_h, ps_h, g_h, idxd, idxs, bufa, bufb, bufc,
                   semi0, semi1, semg0, semg1, semo0, semo1):
        c = lax.axis_index("c")
        s = lax.axis_index("s")
        w = c * 16 + s
        tbase = w * (nch * C)
        tbase2 = tbase // 2
        semi = [semi0, semi1]
        semg = [semg0, semg1]
        semo = [semo0, semo1]

        def issue_idx(k, b):
            base = tbase + k * C
            pltpu.async_copy(dst_h.at[pl.ds(base, C)], idxd.at[b], semi[b])
            pltpu.async_copy(src_h.at[pl.ds(base, C)], idxs.at[b], semi[b])

        def wait_idx(b):
            pltpu.make_async_copy(dst_h.at[pl.ds(0, C)], idxd.at[b], semi[b]).wait()
            pltpu.make_async_copy(src_h.at[pl.ds(0, C)], idxs.at[b], semi[b]).wait()

        def issue_gather(b):
            pltpu.async_copy(pd_h.at[idxd.at[b]], bufa.at[b], semg[b])
            pltpu.async_copy(ps_h.at[idxs.at[b]], bufb.at[b], semg[b])

        def wait_gather(b):
            pltpu.make_async_copy(pd_h.at[idxd.at[b]], bufa.at[b], semg[b]).wait()
            pltpu.make_async_copy(ps_h.at[idxs.at[b]], bufb.at[b], semg[b]).wait()

        def issue_store(k, b):
            pltpu.async_copy(bufc.at[b], g_h.at[pl.ds(tbase2 + k * C2, C2)], semo[b])

        def wait_store(b):
            pltpu.make_async_copy(bufc.at[b], g_h.at[pl.ds(tbase2, C2)], semo[b]).wait()

        def add_rows(b, nrows):
            @plsc.parallel_loop(0, nrows // 2, unroll=4)
            def row(r):
                for p in range(2):
                    for j in range(4):
                        si = pl.ds(j * 16, 16)
                        so = pl.ds(p * 64 + j * 16, 16)
                        bufc[b, r, so] = bufa[b, 2 * r + p, si] + bufb[b, 2 * r + p, si]

        issue_idx(0, 0)
        issue_idx(1, 1)
        wait_idx(0)
        issue_gather(0)
        wait_gather(0)
        issue_idx(2, 0)
        wait_idx(1)
        issue_gather(1)
        add_rows(0, C)
        issue_store(0, 0)
        wait_gather(1)
        issue_idx(3, 1)
        wait_idx(0)
        wait_store(0)
        issue_gather(0)
        add_rows(1, C)
        issue_store(1, 1)

        def pair(kk, carry):
            for b in (0, 1):
                k = 2 + 2 * kk + b
                o = 1 - b
                wait_gather(b)
                issue_idx(k + 2, b)
                wait_idx(o)
                wait_store(o)
                issue_gather(o)
                add_rows(b, C)
                issue_store(k, b)
            return carry

        lax.fori_loop(0, (nch - 4) // 2, pair, 0)
        wait_gather(0)
        wait_idx(1)
        wait_store(1)
        issue_gather(1)
        add_rows(0, C)
        issue_store(nch - 2, 0)
        wait_gather(1)
        add_rows(1, C)
        issue_store(nch - 1, 1)
        wait_store(0)
        wait_store(1)

        if nextra:
            @pl.when(w < nextra)
            def _():
                base = (nch * NW + w) * C
                pltpu.sync_copy(dst_h.at[pl.ds(base, C)], idxd.at[0])
                pltpu.sync_copy(src_h.at[pl.ds(base, C)], idxs.at[0])
                cpa = pltpu.async_copy(pd_h.at[idxd.at[0]], bufa.at[0], semg[0])
                cpb = pltpu.async_copy(ps_h.at[idxs.at[0]], bufb.at[0], semg[0])
                cpa.wait()
                cpb.wait()
                add_rows(0, C)
                pltpu.sync_copy(bufc.at[0], g_h.at[pl.ds(base // 2, C2)])

    return gather_add


def _make_segsum(ne, nch, nextra):
    @functools.partial(
        pl.kernel,
        out_type=jax.ShapeDtypeStruct((2, N, 64), jnp.float32),
        mesh=_mesh,
        scratch_types=[
            pltpu.VMEM((2, C2), jnp.int32),
            pltpu.VMEM((2, C2), jnp.int32),
            pltpu.VMEM((2, C2, 64), jnp.float32),
            pltpu.VMEM((2, C2, 64), jnp.float32),
            pltpu.VMEM_SHARED((N, 64), jnp.float32),
            pltpu.SemaphoreType.DMA,
            pltpu.SemaphoreType.DMA,
            pltpu.SemaphoreType.DMA,
            pltpu.SemaphoreType.DMA,
        ],
        compiler_params=_sc_params,
    )
    def segsum(dste_h, dsto_h, val_h, zeros_h, out_h, idxe, idxo, bufe, bufo,
               accum, seml0, seml1, semc0, semc1):
        c = lax.axis_index("c")
        s = lax.axis_index("s")
        rsl = pl.ds(s * RPT, RPT)
        pltpu.sync_copy(zeros_h.at[rsl], accum.at[rsl])
        plsc.subcore_barrier()
        w = c * 16 + s
        tbase2 = w * (nch * C2)
        seml = [seml0, seml1]
        semc = [semc0, semc1]

        def issue_load(k, b):
            base = tbase2 + k * C2
            pltpu.async_copy(dste_h.at[pl.ds(base, C2)], idxe.at[b], seml[b])
            pltpu.async_copy(dsto_h.at[pl.ds(base, C2)], idxo.at[b], seml[b])
            pltpu.async_copy(val_h.at[pl.ds(base, C2), pl.ds(0, 64)], bufe.at[b], seml[b])
            pltpu.async_copy(val_h.at[pl.ds(base, C2), pl.ds(64, 64)], bufo.at[b], seml[b])

        def wait_load(b):
            pltpu.make_async_copy(dste_h.at[pl.ds(0, C2)], idxe.at[b], seml[b]).wait()
            pltpu.make_async_copy(dsto_h.at[pl.ds(0, C2)], idxo.at[b], seml[b]).wait()
            pltpu.make_async_copy(val_h.at[pl.ds(0, C2), pl.ds(0, 64)], bufe.at[b], seml[b]).wait()
            pltpu.make_async_copy(val_h.at[pl.ds(0, C2), pl.ds(64, 64)], bufo.at[b], seml[b]).wait()

        def issue_scat(b):
            pltpu.async_copy(bufe.at[b], accum.at[idxe.at[b]], semc[b], add=True)
            pltpu.async_copy(bufo.at[b], accum.at[idxo.at[b]], semc[b], add=True)

        def wait_scat(b):
            pltpu.make_async_copy(bufe.at[b], accum.at[idxe.at[b]], semc[b]).wait()
            pltpu.make_async_copy(bufo.at[b], accum.at[idxo.at[b]], semc[b]).wait()

        issue_load(0, 0)
        wait_load(0)
        issue_scat(0)
        issue_load(1, 1)

        def pair(kk, carry):
            for b in (1, 0):
                k = 1 + 2 * kk + (1 - b)
                o = 1 - b
                wait_load(b)
                issue_scat(b)
                wait_scat(o)
                issue_load(k + 1, o)
            return carry

        lax.fori_loop(0, (nch - 2) // 2, pair, 0)
        wait_load(1)
        issue_scat(1)
        wait_scat(0)
        wait_scat(1)

        if nextra:
            @pl.when(w < nextra)
            def _():
                base = (nch * NW + w) * C2
                pltpu.sync_copy(dste_h.at[pl.ds(base, C2)], idxe.at[0])
                pltpu.sync_copy(dsto_h.at[pl.ds(base, C2)], idxo.at[0])
                pltpu.sync_copy(val_h.at[pl.ds(base, C2), pl.ds(0, 64)], bufe.at[0])
                pltpu.sync_copy(val_h.at[pl.ds(base, C2), pl.ds(64, 64)], bufo.at[0])
                pltpu.sync_copy(bufe.at[0], accum.at[idxe.at[0]], add=True)
                pltpu.sync_copy(bufo.at[0], accum.at[idxo.at[0]], add=True)

        plsc.subcore_barrier()
        pltpu.sync_copy(accum.at[rsl], out_h.at[c, rsl])

    return segsum


_gather_a = _make_gather(EA, 38, 0)
_gather_b = _make_gather(EB, 40, 4)
_segsum_a = _make_segsum(EA, 38, 0)
_segsum_b = _make_segsum(EB, 40, 4)


@functools.partial(
    pl.kernel,
    out_type=jax.ShapeDtypeStruct((2, N, 16), jnp.float32),
    mesh=_mesh,
    scratch_types=[
        pltpu.VMEM((2, C), jnp.int32),
        pltpu.VMEM((C, 16), jnp.float32),
        pltpu.VMEM_SHARED((N, 16), jnp.float32),
        pltpu.SemaphoreType.DMA,
        pltpu.SemaphoreType.DMA,
        pltpu.SemaphoreType.DMA,
        pltpu.SemaphoreType.DMA,
    ],
    compiler_params=_sc_params,
)
def _degree(dst_h, ones_h, zeros_h, out_h, idx, buf, accum,
            seml0, seml1, semc0, semc1):
    c = lax.axis_index("c")
    s = lax.axis_index("s")
    rsl = pl.ds(s * RPT, RPT)
    pltpu.sync_copy(zeros_h.at[rsl], accum.at[rsl])
    pltpu.sync_copy(ones_h, buf)
    plsc.subcore_barrier()
    w = c * 16 + s
    nch = 78
    nextra = E // C - nch * NW
    tbase = w * (nch * C)
    seml = [seml0, seml1]
    semc = [semc0, semc1]

    def issue_load(k, b):
        pltpu.async_copy(dst_h.at[pl.ds(tbase + k * C, C)], idx.at[b], seml[b])

    def wait_load(b):
        pltpu.make_async_copy(dst_h.at[pl.ds(0, C)], idx.at[b], seml[b]).wait()

    def issue_scat(b):
        pltpu.async_copy(buf, accum.at[idx.at[b]], semc[b], add=True)

    def wait_scat(b):
        pltpu.make_async_copy(buf, accum.at[idx.at[b]], semc[b]).wait()

    issue_load(0, 0)
    wait_load(0)
    issue_scat(0)
    issue_load(1, 1)

    def pair(kk, carry):
        for b in (1, 0):
            k = 1 + 2 * kk + (1 - b)
            o = 1 - b
            wait_load(b)
            issue_scat(b)
            wait_scat(o)
            issue_load(k + 1, o)
        return carry

    lax.fori_loop(0, (nch - 2) // 2, pair, 0)
    wait_load(1)
    issue_scat(1)
    wait_scat(0)
    wait_scat(1)

    @pl.when(w < nextra)
    def _():
        base = (nch * NW + w) * C
        pltpu.sync_copy(dst_h.at[pl.ds(base, C)], idx.at[0])
        pltpu.sync_copy(buf, accum.at[idx.at[0]], add=True)

    plsc.subcore_barrier()
    pltpu.sync_copy(accum.at[rsl], out_h.at[c, rsl])


TILE_N = 2000
TILE_R = 2048


def _tables_body(x_ref, wd_ref, ws_ref, bd_ref, pd_ref, ps_ref):
    xb = x_ref[...]
    pd_ref[...] = jnp.dot(xb, wd_ref[...], preferred_element_type=jnp.float32) + bd_ref[...]
    ps_ref[...] = jnp.dot(xb, ws_ref[...], preferred_element_type=jnp.float32)


def _mlp_body(g_ref, w1_ref, b1_ref, w2_ref, b2_ref, o_ref):
    gb = g_ref[...]
    h1 = jnp.maximum(jnp.concatenate([gb[:, :64], gb[:, 64:]], axis=0), 0.0)
    h2 = jnp.maximum(
        jnp.dot(h1, w1_ref[...], preferred_element_type=jnp.float32) + b1_ref[...], 0.0)
    h3 = jnp.maximum(
        jnp.dot(h2, w2_ref[...], preferred_element_type=jnp.float32) + b2_ref[...], 0.0)
    half = h3.shape[0] // 2
    o_ref[...] = jnp.concatenate([h3[:half], h3[half:]], axis=1)


def _mid_body(spa_ref, spb_ref, degp_ref, w3_ref, b3_ref, wd2_ref, ws2_ref,
              bd2_ref, pd2_ref, ps2_ref):
    ssum = spa_ref[0] + spa_ref[1] + spb_ref[0] + spb_ref[1]
    deg = degp_ref[0, :, 0:1] + degp_ref[1, :, 0:1]
    h = jnp.maximum(
        jnp.dot(ssum, w3_ref[...], preferred_element_type=jnp.float32) + deg * b3_ref[...],
        0.0)
    pd2_ref[...] = jnp.dot(h, wd2_ref[...], preferred_element_type=jnp.float32) + bd2_ref[...]
    ps2_ref[...] = jnp.dot(h, ws2_ref[...], preferred_element_type=jnp.float32)


def _final_body(spa_ref, spb_ref, degp_ref, w3_ref, b3_ref, o_ref):
    ssum = spa_ref[0] + spa_ref[1] + spb_ref[0] + spb_ref[1]
    deg = degp_ref[0, :, 0:1] + degp_ref[1, :, 0:1]
    o_ref[...] = jnp.dot(ssum, w3_ref[...], preferred_element_type=jnp.float32) + deg * b3_ref[...]


def _tables(x, wd, ws, bd):
    din = x.shape[1]
    return pl.pallas_call(
        _tables_body,
        grid=(N // TILE_N,),
        in_specs=[
            pl.BlockSpec((TILE_N, din), lambda i: (i, 0)),
            pl.BlockSpec((din, 64), lambda i: (0, 0)),
            pl.BlockSpec((din, 64), lambda i: (0, 0)),
            pl.BlockSpec((1, 64), lambda i: (0, 0)),
        ],
        out_specs=[
            pl.BlockSpec((TILE_N, 64), lambda i: (i, 0)),
            pl.BlockSpec((TILE_N, 64), lambda i: (i, 0)),
        ],
        out_shape=[
            jax.ShapeDtypeStruct((N, 64), jnp.float32),
            jax.ShapeDtypeStruct((N, 64), jnp.float32),
        ],
    )(x, wd, ws, bd)


def _mlp(g, w1, b1, w2, b2):
    nr = g.shape[0]
    return pl.pallas_call(
        _mlp_body,
        grid=(pl.cdiv(nr, TILE_R),),
        in_specs=[
            pl.BlockSpec((TILE_R, 128), lambda i: (i, 0)),
            pl.BlockSpec((64, 64), lambda i: (0, 0)),
            pl.BlockSpec((1, 64), lambda i: (0, 0)),
            pl.BlockSpec((64, 64), lambda i: (0, 0)),
            pl.BlockSpec((1, 64), lambda i: (0, 0)),
        ],
        out_specs=pl.BlockSpec((TILE_R, 128), lambda i: (i, 0)),
        out_shape=jax.ShapeDtypeStruct((nr, 128), jnp.float32),
    )(g, w1, b1, w2, b2)


def _mid(spa, spb, degp, w3, b3, wd2, ws2, bd2):
    return pl.pallas_call(
        _mid_body,
        grid=(N // TILE_N,),
        in_specs=[
            pl.BlockSpec((2, TILE_N, 64), lambda i: (0, i, 0)),
            pl.BlockSpec((2, TILE_N, 64), lambda i: (0, i, 0)),
            pl.BlockSpec((2, TILE_N, 16), lambda i: (0, i, 0)),
            pl.BlockSpec((64, 64), lambda i: (0, 0)),
            pl.BlockSpec((1, 64), lambda i: (0, 0)),
            pl.BlockSpec((64, 64), lambda i: (0, 0)),
            pl.BlockSpec((64, 64), lambda i: (0, 0)),
            pl.BlockSpec((1, 64), lambda i: (0, 0)),
        ],
        out_specs=[
            pl.BlockSpec((TILE_N, 64), lambda i: (i, 0)),
            pl.BlockSpec((TILE_N, 64), lambda i: (i, 0)),
        ],
        out_shape=[
            jax.ShapeDtypeStruct((N, 64), jnp.float32),
            jax.ShapeDtypeStruct((N, 64), jnp.float32),
        ],
    )(spa, spb, degp, w3, b3, wd2, ws2, bd2)


def _final(spa, spb, degp, w3, b3):
    return pl.pallas_call(
        _final_body,
        grid=(N // TILE_N,),
        in_specs=[
            pl.BlockSpec((2, TILE_N, 64), lambda i: (0, i, 0)),
            pl.BlockSpec((2, TILE_N, 64), lambda i: (0, i, 0)),
            pl.BlockSpec((2, TILE_N, 16), lambda i: (0, i, 0)),
            pl.BlockSpec((64, 128), lambda i: (0, 0)),
            pl.BlockSpec((1, 128), lambda i: (0, 0)),
        ],
        out_specs=pl.BlockSpec((TILE_N, 128), lambda i: (i, 0)),
        out_shape=jax.ShapeDtypeStruct((N, 128), jnp.float32),
    )(spa, spb, degp, w3, b3)


def _conv(src, dst, tabs, w1, b1, w2, b2, zeros64):
    pd, ps = tabs
    sps = []
    for lo, hi, gat, seg in ((0, EA, _gather_a, _segsum_a),
                             (EA, E, _gather_b, _segsum_b)):
        d = dst[lo:hi]
        g = gat(d, src[lo:hi], pd, ps)
        h3 = _mlp(g, w1, b1, w2, b2)
        sps.append(seg(d[0::2], d[1::2], h3, zeros64))
    return sps


def kernel(x, edge_index, params):
    src = edge_index[0]
    dst = edge_index[1]
    (W0, b0), (W1, b1), (W2, b2), (W3, b3) = params["conv1"]
    (V0, c0), (V1, c1), (V2, c2), (V3, c3) = params["conv2"]
    W0i, W0j = W0[:128], W0[128:]
    V0i, V0j = V0[:64], V0[64:]

    zeros64 = jnp.zeros((N, 64), jnp.float32)
    zeros16 = jnp.zeros((N, 16), jnp.float32)
    onesC = jnp.ones((C, 16), jnp.float32)

    degp = _degree(dst, onesC, zeros16)

    tabs1 = _tables(x, W0i - W0j, W0j, b0.reshape(1, 64))
    s1a, s1b = _conv(src, dst, tabs1, W1, b1.reshape(1, 64),
                     W2, b2.reshape(1, 64), zeros64)

    tabs2 = _mid(s1a, s1b, degp, W3, b3.reshape(1, 64),
                 V0i - V0j, V0j, c0.reshape(1, 64))
    s2a, s2b = _conv(src, dst, tabs2, V1, c1.reshape(1, 64),
                     V2, c2.reshape(1, 64), zeros64)

    return _final(s2a, s2b, degp, V3, c3.reshape(1, 128))
